# baseline (speedup 1.0000x reference)
import functools

import jax
import jax.numpy as jnp
from jax import lax
from jax.experimental import pallas as pl
from jax.experimental.pallas import tpu as pltpu
from jax.experimental.pallas import tpu_sc as plsc

D_MODEL = 768
D_FF = 2048
N_EXP = 8
TOKENS = 2048
NPAIR = 2 * TOKENS
TB = 512
TBSHIFT = 9
NB = NPAIR // TB + N_EXP
NBTB = NB * TB


def _router_body(x_ref, wr_ref, pos0_ref, pos1_ref, p0_ref, p1_ref,
                 etab_ref, nblk_ref, xpack_ref):
    x = x_ref[...]
    bits = lax.bitcast_convert_type(x, jnp.int32)
    rne = (bits + 0x7FFF + ((bits >> 16) & 1)) >> 16
    lo = rne[:, : D_MODEL // 2] & 0xFFFF
    hi = rne[:, D_MODEL // 2:] << 16
    xpack_ref[...] = lo | hi
    logits = lax.dot_general(x, wr_ref[...], (((1,), (0,)), ((), ())),
                             preferred_element_type=jnp.float32)
    m = jnp.max(logits, axis=-1, keepdims=True)
    p = jnp.exp(logits - m)
    probs = p / jnp.sum(p, axis=-1, keepdims=True)
    lane = lax.broadcasted_iota(jnp.int32, probs.shape, 1)
    big = jnp.int32(1 << 20)
    m0 = jnp.max(probs, axis=-1, keepdims=True)
    i0 = jnp.min(jnp.where(probs == m0, lane, big), axis=-1, keepdims=True)
    probs2 = jnp.where(lane == i0, -1.0, probs)
    m1 = jnp.max(probs2, axis=-1, keepdims=True)
    i1 = jnp.min(jnp.where(probs2 == m1, lane, big), axis=-1, keepdims=True)
    den = m0 + m1
    p0_ref[...] = m0 / den
    p1_ref[...] = m1 / den

    oh0 = (lane == i0).astype(jnp.float32)
    oh1 = (lane == i1).astype(jnp.float32)
    counts = (jnp.sum(oh0, axis=0, keepdims=True)
              + jnp.sum(oh1, axis=0, keepdims=True))
    nb_i = (counts.astype(jnp.int32) + (TB - 1)) >> TBSHIFT
    nb = nb_i.astype(jnp.float32)
    e_r = lax.broadcasted_iota(jnp.int32, (N_EXP, N_EXP), 0)
    e_c = lax.broadcasted_iota(jnp.int32, (N_EXP, N_EXP), 1)
    ut = (e_r < e_c).astype(jnp.float32)
    bb = lax.dot_general(nb, ut, (((1,), (0,)), ((), ())),
                         preferred_element_type=jnp.float32)
    total = bb[:, N_EXP - 1:] + nb[:, N_EXP - 1:]
    off = bb * TB
    CH = 256
    r = lax.broadcasted_iota(jnp.int32, (CH, CH), 0)
    c = lax.broadcasted_iota(jnp.int32, (CH, CH), 1)
    tri = (c < r).astype(jnp.float32)
    run = jnp.zeros((1, N_EXP), jnp.float32)
    for cc in range(TOKENS // CH):
        sl = slice(cc * CH, (cc + 1) * CH)
        ohc = oh0[sl, :]
        csc = lax.dot_general(tri, ohc, (((1,), (0,)), ((), ())),
                              preferred_element_type=jnp.float32) + run
        pos0_ref[sl, :] = jnp.sum(ohc * (off + csc), axis=-1,
                                  keepdims=True).astype(jnp.int32)
        run = run + jnp.sum(ohc, axis=0, keepdims=True)
    for cc in range(TOKENS // CH):
        sl = slice(cc * CH, (cc + 1) * CH)
        ohc = oh1[sl, :]
        csc = lax.dot_general(tri, ohc, (((1,), (0,)), ((), ())),
                              preferred_element_type=jnp.float32) + run
        pos1_ref[sl, :] = jnp.sum(ohc * (off + csc), axis=-1,
                                  keepdims=True).astype(jnp.int32)
        run = run + jnp.sum(ohc, axis=0, keepdims=True)
    b_col = lax.broadcasted_iota(jnp.int32, (NB, N_EXP), 0).astype(jnp.float32)
    b_eff = jnp.minimum(b_col, total - 1.0)
    beb = jnp.sum((bb <= b_eff).astype(jnp.int32), axis=-1,
                  keepdims=True) - 1
    etab_ref[...] = beb
    nblk_ref[...] = total.astype(jnp.int32)


def _router(xf, Wr):
    return pl.pallas_call(
        _router_body,
        out_shape=(
            jax.ShapeDtypeStruct((TOKENS, 1), jnp.int32),
            jax.ShapeDtypeStruct((TOKENS, 1), jnp.int32),
            jax.ShapeDtypeStruct((TOKENS, 1), jnp.float32),
            jax.ShapeDtypeStruct((TOKENS, 1), jnp.float32),
            jax.ShapeDtypeStruct((NB, 1), jnp.int32),
            jax.ShapeDtypeStruct((1, 1), jnp.int32),
            jax.ShapeDtypeStruct((TOKENS, D_MODEL // 2), jnp.int32),
        ),
        compiler_params=pltpu.CompilerParams(vmem_limit_bytes=60 * 1024 * 1024),
    )(xf, Wr)


NC, NS = 2, 16
NW = NC * NS
TPW = TOKENS // NW


def _dispatch(xf, pos0, pos1):
    mesh = plsc.VectorSubcoreMesh(core_axis_name="c", subcore_axis_name="s")

    @functools.partial(
        pl.kernel, mesh=mesh,
        out_type=jax.ShapeDtypeStruct((NBTB, D_MODEL // 2), jnp.int32),
        scratch_types=[
            pltpu.VMEM((TPW, D_MODEL // 2), jnp.int32),
            pltpu.VMEM((TPW,), jnp.int32),
            pltpu.VMEM((TPW,), jnp.int32),
            pltpu.SemaphoreType.DMA,
            pltpu.SemaphoreType.DMA,
        ],
        compiler_params=pltpu.CompilerParams(needs_layout_passes=False),
    )
    def k(x_hbm, pos0_hbm, pos1_hbm, xs_hbm, rows_v, idx0_v, idx1_v,
          sem0, sem1):
        wid = lax.axis_index("s") * NC + lax.axis_index("c")
        base = wid * TPW
        pltpu.sync_copy(x_hbm.at[pl.ds(base, TPW)], rows_v)
        pltpu.sync_copy(pos0_hbm.at[pl.ds(base, TPW)], idx0_v)
        pltpu.sync_copy(pos1_hbm.at[pl.ds(base, TPW)], idx1_v)
        c0 = pltpu.async_copy(rows_v, xs_hbm.at[idx0_v], sem0)
        c1 = pltpu.async_copy(rows_v, xs_hbm.at[idx1_v], sem1)
        c0.wait()
        c1.wait()

    return k(xf, pos0, pos1)


def _expert_body(etab_ref, nblk_ref, xs_ref, wg_ref, wu_ref, wd_ref,
                 y_ref):
    b = pl.program_id(0)
    n = nblk_ref[0]

    @pl.when(b < n)
    def _compute():
        wg = wg_ref[0].astype(jnp.bfloat16)
        wu = wu_ref[0].astype(jnp.bfloat16)
        wd = wd_ref[0].astype(jnp.bfloat16)
        xq = xs_ref[...]
        lo_f = lax.bitcast_convert_type(xq << 16, jnp.float32)
        hi_f = lax.bitcast_convert_type(xq & jnp.int32(-65536), jnp.float32)
        xb = jnp.concatenate([lo_f, hi_f], axis=1).astype(jnp.bfloat16)
        g = lax.dot_general(xb, wg, (((1,), (0,)), ((), ())),
                            preferred_element_type=jnp.float32)
        u = lax.dot_general(xb, wu, (((1,), (0,)), ((), ())),
                            preferred_element_type=jnp.float32)
        h = ((g / (1.0 + jnp.exp(-g))) * u).astype(jnp.bfloat16)
        eo = lax.dot_general(h, wd, (((1,), (0,)), ((), ())),
                             preferred_element_type=jnp.float32)
        bits = lax.bitcast_convert_type(eo, jnp.int32)
        rne = (bits + 0x7FFF + ((bits >> 16) & 1)) >> 16
        y_ref[...] = (rne[:, : D_MODEL // 2] & 0xFFFF) | (
            rne[:, D_MODEL // 2:] << 16)


def _experts(x_sorted, etab, nblk, W_gate, W_up, W_down):
    grid_spec = pltpu.PrefetchScalarGridSpec(
        num_scalar_prefetch=2,
        grid=(NB,),
        in_specs=[
            pl.BlockSpec((TB, D_MODEL // 2),
                         lambda b, etab, nblk: (jnp.minimum(b, nblk[0] - 1), 0)),
            pl.BlockSpec((1, D_MODEL, D_FF),
                         lambda b, etab, nblk: (etab[b], 0, 0)),
            pl.BlockSpec((1, D_MODEL, D_FF),
                         lambda b, etab, nblk: (etab[b], 0, 0)),
            pl.BlockSpec((1, D_FF, D_MODEL),
                         lambda b, etab, nblk: (etab[b], 0, 0)),
        ],
        out_specs=pl.BlockSpec((TB, D_MODEL // 2),
                               lambda b, etab, nblk:
                               (jnp.minimum(b, nblk[0] - 1), 0)),
    )
    return pl.pallas_call(
        _expert_body,
        grid_spec=grid_spec,
        out_shape=jax.ShapeDtypeStruct((NBTB, D_MODEL // 2), jnp.int32),
        compiler_params=pltpu.CompilerParams(vmem_limit_bytes=60 * 1024 * 1024),
    )(etab, nblk, x_sorted, W_gate, W_up, W_down)


def _combine(y_sorted, pos0, pos1, p0, p1):
    mesh = plsc.VectorSubcoreMesh(core_axis_name="c", subcore_axis_name="s")

    @functools.partial(
        pl.kernel, mesh=mesh,
        out_type=jax.ShapeDtypeStruct((TOKENS, D_MODEL), jnp.float32),
        scratch_types=[
            pltpu.VMEM((TPW, D_MODEL // 2), jnp.int32),
            pltpu.VMEM((TPW, D_MODEL // 2), jnp.int32),
            pltpu.VMEM((TPW, D_MODEL), jnp.float32),
            pltpu.VMEM((TPW,), jnp.int32),
            pltpu.VMEM((TPW,), jnp.int32),
            pltpu.VMEM((TPW,), jnp.float32),
            pltpu.VMEM((TPW,), jnp.float32),
            pltpu.SemaphoreType.DMA,
            pltpu.SemaphoreType.DMA,
        ],
        compiler_params=pltpu.CompilerParams(needs_layout_passes=False),
    )
    def k(y_hbm, pos0_hbm, pos1_hbm, p0_hbm, p1_hbm, out_hbm,
          r0_v, r1_v, o_v, idx0_v, idx1_v, p0_v, p1_v, sem0, sem1):
        wid = lax.axis_index("s") * NC + lax.axis_index("c")
        base = wid * TPW
        pltpu.sync_copy(pos0_hbm.at[pl.ds(base, TPW)], idx0_v)
        pltpu.sync_copy(pos1_hbm.at[pl.ds(base, TPW)], idx1_v)
        pltpu.sync_copy(p0_hbm.at[pl.ds(base, TPW)], p0_v)
        pltpu.sync_copy(p1_hbm.at[pl.ds(base, TPW)], p1_v)
        c0 = pltpu.async_copy(y_hbm.at[idx0_v], r0_v, sem0)
        c1 = pltpu.async_copy(y_hbm.at[idx1_v], r1_v, sem1)
        c0.wait()
        c1.wait()

        himask = jnp.full((16,), -65536, jnp.int32)

        def body(rr, carry):
            lanes = jnp.full((16,), rr, jnp.int32)
            v0 = plsc.load_gather(p0_v, [lanes])
            v1 = plsc.load_gather(p1_v, [lanes])
            for j in range(D_MODEL // 32):
                sl = pl.ds(j * 16, 16)
                sh = pl.ds(j * 16 + D_MODEL // 2, 16)
                q0 = r0_v[rr, sl]
                q1 = r1_v[rr, sl]
                lo0 = plsc.bitcast(q0 << 16, jnp.float32)
                lo1 = plsc.bitcast(q1 << 16, jnp.float32)
                hi0 = plsc.bitcast(q0 & himask, jnp.float32)
                hi1 = plsc.bitcast(q1 & himask, jnp.float32)
                o_v[rr, sl] = v0 * lo0 + v1 * lo1
                o_v[rr, sh] = v0 * hi0 + v1 * hi1
            return carry

        lax.fori_loop(0, TPW, body, 0)
        pltpu.sync_copy(o_v, out_hbm.at[pl.ds(base, TPW)])

    return k(y_sorted, pos0, pos1, p0, p1)


def kernel(x, Wr, W_gate, W_up, W_down):
    B, S, D = x.shape
    xf = x.reshape(-1, D)
    pos0, pos1, p0, p1, etab, nblk, xpack = _router(xf, Wr)
    pos0f = pos0.reshape(-1)
    pos1f = pos1.reshape(-1)
    x_sorted = _dispatch(xpack, pos0f, pos1f)
    y_sorted = _experts(x_sorted, etab.reshape(-1),
                        nblk.reshape(-1), W_gate, W_up, W_down)
    out = _combine(y_sorted, pos0f, pos1f, p0.reshape(-1), p1.reshape(-1))
    return out.reshape(B, S, D)


# speedup vs baseline: 1.0429x; 1.0429x over previous
import functools

import jax
import jax.numpy as jnp
from jax import lax
from jax.experimental import pallas as pl
from jax.experimental.pallas import tpu as pltpu
from jax.experimental.pallas import tpu_sc as plsc

D_MODEL = 768
D_FF = 2048
N_EXP = 8
TOKENS = 2048
NPAIR = 2 * TOKENS
TB = 512
TBSHIFT = 9
NB = NPAIR // TB + N_EXP
NBTB = NB * TB
TCH = 256


def _router_body(x_ref, wr_ref, pos0_ref, pos1_ref, p0_ref, p1_ref,
                 etab_ref, nblk_ref, rcnt_ref, xpack_ref):
    x = x_ref[...]
    bits = lax.bitcast_convert_type(x, jnp.int32)
    rne = (bits + 0x7FFF + ((bits >> 16) & 1)) >> 16
    lo = rne[:, : D_MODEL // 2] & 0xFFFF
    hi = rne[:, D_MODEL // 2:] << 16
    xpack_ref[...] = lo | hi
    logits = lax.dot_general(x, wr_ref[...], (((1,), (0,)), ((), ())),
                             preferred_element_type=jnp.float32)
    m = jnp.max(logits, axis=-1, keepdims=True)
    p = jnp.exp(logits - m)
    probs = p / jnp.sum(p, axis=-1, keepdims=True)
    lane = lax.broadcasted_iota(jnp.int32, probs.shape, 1)
    big = jnp.int32(1 << 20)
    m0 = jnp.max(probs, axis=-1, keepdims=True)
    i0 = jnp.min(jnp.where(probs == m0, lane, big), axis=-1, keepdims=True)
    probs2 = jnp.where(lane == i0, -1.0, probs)
    m1 = jnp.max(probs2, axis=-1, keepdims=True)
    i1 = jnp.min(jnp.where(probs2 == m1, lane, big), axis=-1, keepdims=True)
    den = m0 + m1
    p0_ref[...] = m0 / den
    p1_ref[...] = m1 / den

    oh0 = (lane == i0).astype(jnp.float32)
    oh1 = (lane == i1).astype(jnp.float32)
    counts = (jnp.sum(oh0, axis=0, keepdims=True)
              + jnp.sum(oh1, axis=0, keepdims=True))
    nb_i = (counts.astype(jnp.int32) + (TB - 1)) >> TBSHIFT
    nb = nb_i.astype(jnp.float32)
    e_r = lax.broadcasted_iota(jnp.int32, (N_EXP, N_EXP), 0)
    e_c = lax.broadcasted_iota(jnp.int32, (N_EXP, N_EXP), 1)
    ut = (e_r < e_c).astype(jnp.float32)
    bb = lax.dot_general(nb, ut, (((1,), (0,)), ((), ())),
                         preferred_element_type=jnp.float32)
    total = bb[:, N_EXP - 1:] + nb[:, N_EXP - 1:]
    off = bb * TB
    CH = 256
    r = lax.broadcasted_iota(jnp.int32, (CH, CH), 0)
    c = lax.broadcasted_iota(jnp.int32, (CH, CH), 1)
    tri = (c < r).astype(jnp.float32)
    run = jnp.zeros((1, N_EXP), jnp.float32)
    for cc in range(TOKENS // CH):
        sl = slice(cc * CH, (cc + 1) * CH)
        ohc = oh0[sl, :]
        csc = lax.dot_general(tri, ohc, (((1,), (0,)), ((), ())),
                              preferred_element_type=jnp.float32) + run
        pos0_ref[sl, :] = jnp.sum(ohc * (off + csc), axis=-1,
                                  keepdims=True).astype(jnp.int32)
        run = run + jnp.sum(ohc, axis=0, keepdims=True)
    for cc in range(TOKENS // CH):
        sl = slice(cc * CH, (cc + 1) * CH)
        ohc = oh1[sl, :]
        csc = lax.dot_general(tri, ohc, (((1,), (0,)), ((), ())),
                              preferred_element_type=jnp.float32) + run
        pos1_ref[sl, :] = jnp.sum(ohc * (off + csc), axis=-1,
                                  keepdims=True).astype(jnp.int32)
        run = run + jnp.sum(ohc, axis=0, keepdims=True)
    b_col = lax.broadcasted_iota(jnp.int32, (NB, N_EXP), 0).astype(jnp.float32)
    b_eff = jnp.minimum(b_col, total - 1.0)
    beb = jnp.sum((bb <= b_eff).astype(jnp.int32), axis=-1,
                  keepdims=True) - 1
    etab_ref[...] = beb
    nblk_ref[...] = total.astype(jnp.int32)
    e_lane = lax.broadcasted_iota(jnp.int32, (NB, N_EXP), 1)
    ohb = (e_lane == beb).astype(jnp.float32)
    bb_b = jnp.sum(ohb * bb, axis=-1, keepdims=True)
    cnt_b = jnp.sum(ohb * counts, axis=-1, keepdims=True)
    rows_b = jnp.minimum(cnt_b - (b_col[:, :1] - bb_b) * TB, float(TB))
    rcnt_ref[...] = rows_b.astype(jnp.int32)


def _router(xf, Wr):
    return pl.pallas_call(
        _router_body,
        out_shape=(
            jax.ShapeDtypeStruct((TOKENS, 1), jnp.int32),
            jax.ShapeDtypeStruct((TOKENS, 1), jnp.int32),
            jax.ShapeDtypeStruct((TOKENS, 1), jnp.float32),
            jax.ShapeDtypeStruct((TOKENS, 1), jnp.float32),
            jax.ShapeDtypeStruct((NB, 1), jnp.int32),
            jax.ShapeDtypeStruct((1, 1), jnp.int32),
            jax.ShapeDtypeStruct((NB, 1), jnp.int32),
            jax.ShapeDtypeStruct((TOKENS, D_MODEL // 2), jnp.int32),
        ),
        compiler_params=pltpu.CompilerParams(vmem_limit_bytes=60 * 1024 * 1024),
    )(xf, Wr)


NC, NS = 2, 16
NW = NC * NS
TPW = TOKENS // NW


def _dispatch(xf, pos0, pos1):
    mesh = plsc.VectorSubcoreMesh(core_axis_name="c", subcore_axis_name="s")

    @functools.partial(
        pl.kernel, mesh=mesh,
        out_type=jax.ShapeDtypeStruct((NBTB, D_MODEL // 2), jnp.int32),
        scratch_types=[
            pltpu.VMEM((TPW, D_MODEL // 2), jnp.int32),
            pltpu.VMEM((TPW,), jnp.int32),
            pltpu.VMEM((TPW,), jnp.int32),
            pltpu.SemaphoreType.DMA,
            pltpu.SemaphoreType.DMA,
        ],
        compiler_params=pltpu.CompilerParams(needs_layout_passes=False),
    )
    def k(x_hbm, pos0_hbm, pos1_hbm, xs_hbm, rows_v, idx0_v, idx1_v,
          sem0, sem1):
        wid = lax.axis_index("s") * NC + lax.axis_index("c")
        base = wid * TPW
        pltpu.sync_copy(x_hbm.at[pl.ds(base, TPW)], rows_v)
        pltpu.sync_copy(pos0_hbm.at[pl.ds(base, TPW)], idx0_v)
        pltpu.sync_copy(pos1_hbm.at[pl.ds(base, TPW)], idx1_v)
        c0 = pltpu.async_copy(rows_v, xs_hbm.at[idx0_v], sem0)
        c1 = pltpu.async_copy(rows_v, xs_hbm.at[idx1_v], sem1)
        c0.wait()
        c1.wait()

    return k(xf, pos0, pos1)


def _expert_body(etab_ref, nblk_ref, rcnt_ref, xs_ref, wg_ref, wu_ref, wd_ref,
                 y_ref):
    b = pl.program_id(0)
    n = nblk_ref[0]

    @pl.when(b < n)
    def _compute():
        wg = wg_ref[0].astype(jnp.bfloat16)
        wu = wu_ref[0].astype(jnp.bfloat16)
        wd = wd_ref[0].astype(jnp.bfloat16)
        r = rcnt_ref[b]
        for cc in range(TB // TCH):

            @pl.when(jnp.logical_or(cc == 0, r > cc * TCH))
            def _chunk():
                sl = slice(cc * TCH, (cc + 1) * TCH)
                xq = xs_ref[sl, :]
                lo_f = lax.bitcast_convert_type(xq << 16, jnp.float32)
                hi_f = lax.bitcast_convert_type(xq & jnp.int32(-65536),
                                                jnp.float32)
                xb = jnp.concatenate([lo_f, hi_f], axis=1).astype(jnp.bfloat16)
                g = lax.dot_general(xb, wg, (((1,), (0,)), ((), ())),
                                    preferred_element_type=jnp.float32)
                u = lax.dot_general(xb, wu, (((1,), (0,)), ((), ())),
                                    preferred_element_type=jnp.float32)
                h = ((g / (1.0 + jnp.exp(-g))) * u).astype(jnp.bfloat16)
                eo = lax.dot_general(h, wd, (((1,), (0,)), ((), ())),
                                     preferred_element_type=jnp.float32)
                y_ref[sl, :] = eo


def _experts(x_sorted, etab, nblk, rcnt, W_gate, W_up, W_down):
    grid_spec = pltpu.PrefetchScalarGridSpec(
        num_scalar_prefetch=3,
        grid=(NB,),
        in_specs=[
            pl.BlockSpec((TB, D_MODEL // 2),
                         lambda b, etab, nblk, rcnt:
                         (jnp.minimum(b, nblk[0] - 1), 0)),
            pl.BlockSpec((1, D_MODEL, D_FF),
                         lambda b, etab, nblk, rcnt: (etab[b], 0, 0)),
            pl.BlockSpec((1, D_MODEL, D_FF),
                         lambda b, etab, nblk, rcnt: (etab[b], 0, 0)),
            pl.BlockSpec((1, D_FF, D_MODEL),
                         lambda b, etab, nblk, rcnt: (etab[b], 0, 0)),
        ],
        out_specs=pl.BlockSpec((TB, D_MODEL),
                               lambda b, etab, nblk, rcnt:
                               (jnp.minimum(b, nblk[0] - 1), 0)),
    )
    return pl.pallas_call(
        _expert_body,
        grid_spec=grid_spec,
        out_shape=jax.ShapeDtypeStruct((NBTB, D_MODEL), jnp.float32),
        compiler_params=pltpu.CompilerParams(vmem_limit_bytes=60 * 1024 * 1024),
    )(etab, nblk, rcnt, x_sorted, W_gate, W_up, W_down)


def _combine(y_sorted, pos0, pos1, p0, p1):
    mesh = plsc.VectorSubcoreMesh(core_axis_name="c", subcore_axis_name="s")

    @functools.partial(
        pl.kernel, mesh=mesh,
        out_type=jax.ShapeDtypeStruct((TOKENS, D_MODEL), jnp.float32),
        scratch_types=[
            pltpu.VMEM((TPW, D_MODEL), jnp.float32),
            pltpu.VMEM((TPW, D_MODEL), jnp.float32),
            pltpu.VMEM((TPW,), jnp.int32),
            pltpu.VMEM((TPW,), jnp.int32),
            pltpu.VMEM((TPW,), jnp.float32),
            pltpu.VMEM((TPW,), jnp.float32),
            pltpu.SemaphoreType.DMA,
            pltpu.SemaphoreType.DMA,
        ],
        compiler_params=pltpu.CompilerParams(needs_layout_passes=False),
    )
    def k(y_hbm, pos0_hbm, pos1_hbm, p0_hbm, p1_hbm, out_hbm,
          r0_v, r1_v, idx0_v, idx1_v, p0_v, p1_v, sem0, sem1):
        wid = lax.axis_index("s") * NC + lax.axis_index("c")
        base = wid * TPW
        pltpu.sync_copy(pos0_hbm.at[pl.ds(base, TPW)], idx0_v)
        pltpu.sync_copy(pos1_hbm.at[pl.ds(base, TPW)], idx1_v)
        pltpu.sync_copy(p0_hbm.at[pl.ds(base, TPW)], p0_v)
        pltpu.sync_copy(p1_hbm.at[pl.ds(base, TPW)], p1_v)
        c0 = pltpu.async_copy(y_hbm.at[idx0_v], r0_v, sem0)
        c1 = pltpu.async_copy(y_hbm.at[idx1_v], r1_v, sem1)
        c0.wait()
        c1.wait()

        def body(rr, carry):
            lanes = jnp.full((16,), rr, jnp.int32)
            v0 = plsc.load_gather(p0_v, [lanes])
            v1 = plsc.load_gather(p1_v, [lanes])
            for j in range(D_MODEL // 16):
                sl = pl.ds(j * 16, 16)
                r0_v[rr, sl] = v0 * r0_v[rr, sl] + v1 * r1_v[rr, sl]
            return carry

        lax.fori_loop(0, TPW, body, 0)
        pltpu.sync_copy(r0_v, out_hbm.at[pl.ds(base, TPW)])

    return k(y_sorted, pos0, pos1, p0, p1)


def kernel(x, Wr, W_gate, W_up, W_down):
    B, S, D = x.shape
    xf = x.reshape(-1, D)
    pos0, pos1, p0, p1, etab, nblk, rcnt, xpack = _router(xf, Wr)
    pos0f = pos0.reshape(-1)
    pos1f = pos1.reshape(-1)
    x_sorted = _dispatch(xpack, pos0f, pos1f)
    y_sorted = _experts(x_sorted, etab.reshape(-1), nblk.reshape(-1),
                        rcnt.reshape(-1), W_gate, W_up, W_down)
    out = _combine(y_sorted, pos0f, pos1f, p0.reshape(-1), p1.reshape(-1))
    return out.reshape(B, S, D)
